# baseline (speedup 1.0000x reference)
import functools

import jax
import jax.numpy as jnp
from jax import lax
from jax.experimental import pallas as pl
from jax.experimental.pallas import tpu as pltpu
from jax.experimental.pallas import tpu_sc as plsc

_B = 2048
_N = 64
_H = 128
_FEAT = 512
_L = 16
_NC = 2
_NS = 16
_NW = _NC * _NS
_UPT = _B // _NW
_HV = _H // _L


def _sc_reduce_body(xi_hbm, xt_hbm, oi_hbm, ot_hbm, buf_i, buf_t, out_i, out_t):
    wid = lax.axis_index("s") * _NC + lax.axis_index("c")
    base = wid * _UPT

    def one_user(u, buf, hbm, out_v):
        pltpu.sync_copy(hbm.at[base + u], buf)

        def row_body(r, accs):
            return tuple(accs[h] + buf[r, pl.ds(h * _L, _L)]
                         for h in range(_HV))

        accs = lax.fori_loop(
            0, _N, row_body,
            tuple(jnp.zeros((_L,), jnp.float32) for _ in range(_HV)))
        for h in range(_HV):
            out_v[u, pl.ds(h * _L, _L)] = accs[h]

    def user_body(u, carry):
        one_user(u, buf_i, xi_hbm, out_i)
        one_user(u, buf_t, xt_hbm, out_t)
        return carry

    lax.fori_loop(0, _UPT, user_body, 0)
    pltpu.sync_copy(out_i, oi_hbm.at[pl.ds(base, _UPT)])
    pltpu.sync_copy(out_t, ot_hbm.at[pl.ds(base, _UPT)])


def _sc_reduce(xi, xt):
    mesh = plsc.VectorSubcoreMesh(core_axis_name="c", subcore_axis_name="s")
    f = functools.partial(
        pl.kernel,
        out_type=(jax.ShapeDtypeStruct((_B, _H), jnp.float32),
                  jax.ShapeDtypeStruct((_B, _H), jnp.float32)),
        mesh=mesh,
        scratch_types=[
            pltpu.VMEM((_N, _H), jnp.float32),
            pltpu.VMEM((_N, _H), jnp.float32),
            pltpu.VMEM((_UPT, _H), jnp.float32),
            pltpu.VMEM((_UPT, _H), jnp.float32),
        ],
    )(_sc_reduce_body)
    return f(xi, xt)


def _tc_tail_body(feat_ref, si_ref, st_ref, wu_ref, bu_ref, wli_ref, wlt_ref,
                  wri_ref, wrt_ref, bli_ref, blt_ref, out_ref):
    inv_n = jnp.float32(1.0 / _N)
    dn = (((1,), (1,)), ((), ()))
    user = jax.lax.dot_general(feat_ref[...], wu_ref[...], dn,
                               preferred_element_type=jnp.float32)
    user = user + bu_ref[...]
    acc = jax.lax.dot_general(si_ref[...] * inv_n, wli_ref[...], dn,
                              preferred_element_type=jnp.float32)
    acc = acc + jax.lax.dot_general(st_ref[...] * inv_n, wlt_ref[...], dn,
                                    preferred_element_type=jnp.float32)
    wr = wri_ref[...] + wrt_ref[...]
    acc = acc + jax.lax.dot_general(user, wr, dn,
                                    preferred_element_type=jnp.float32)
    acc = acc + bli_ref[...] + blt_ref[...]
    out_ref[...] = jnp.maximum(acc, 0.0)


@jax.jit
def kernel(input_text, input_img, base_text_features, base_img_features,
           W_user, b_user, Wl_img, bl_img, Wr_img, Wl_txt, bl_txt, Wr_txt):
    feat = jnp.concatenate([input_text[:, 0, :], input_img[:, 0, :]], axis=1)
    sum_i, sum_t = _sc_reduce(base_img_features, base_text_features)
    out = pl.pallas_call(
        _tc_tail_body,
        out_shape=jax.ShapeDtypeStruct((_B, _H), jnp.float32),
    )(feat, sum_i, sum_t,
      W_user, b_user.reshape(1, _H), Wl_img, Wl_txt, Wr_img, Wr_txt,
      bl_img.reshape(1, _H), bl_txt.reshape(1, _H))
    return out


# speedup vs baseline: 2.2704x; 2.2704x over previous
import functools

import jax
import jax.numpy as jnp
from jax import lax
from jax.experimental import pallas as pl
from jax.experimental.pallas import tpu as pltpu
from jax.experimental.pallas import tpu_sc as plsc

_B = 2048
_N = 64
_H = 128
_FEAT = 512
_L = 16
_NC = 2
_NS = 16
_NW = _NC * _NS
_UPT = _B // _NW
_HV = _H // _L


_CU = 2
_NCH = _UPT // _CU
_RU = 8


def _sc_reduce_body(xi_hbm, xt_hbm, oi_hbm, ot_hbm, buf_i, buf_t,
                    out_i, out_t, si0, si1, st0, st1):
    wid = lax.axis_index("s") * _NC + lax.axis_index("c")
    base = wid * _UPT
    sem_i = (si0, si1)
    sem_t = (st0, st1)

    def dma_in(hbm, buf, sem, c, slot):
        return pltpu.make_async_copy(
            hbm.at[pl.ds(base + c * _CU, _CU)], buf.at[slot], sem)

    def acc_user(buf, slot, u, out_v, row):
        def g_body(g, accs):
            r0 = g * _RU
            for k in range(_RU):
                accs = tuple(accs[h] + buf[slot, u, r0 + k, pl.ds(h * _L, _L)]
                             for h in range(_HV))
            return accs

        accs = lax.fori_loop(
            0, _N // _RU, g_body,
            tuple(jnp.zeros((_L,), jnp.float32) for _ in range(_HV)))
        for h in range(_HV):
            out_v[row, pl.ds(h * _L, _L)] = accs[h]

    def do_chunk(c, slot):
        dma_in(xi_hbm, buf_i, sem_i[slot], c, slot).wait()
        dma_in(xt_hbm, buf_t, sem_t[slot], c, slot).wait()
        for u in range(_CU):
            acc_user(buf_i, slot, u, out_i, c * _CU + u)
            acc_user(buf_t, slot, u, out_t, c * _CU + u)

    dma_in(xi_hbm, buf_i, sem_i[0], 0, 0).start()
    dma_in(xt_hbm, buf_t, sem_t[0], 0, 0).start()

    def pair_body(p, carry):
        c0 = p * 2
        c1 = c0 + 1
        dma_in(xi_hbm, buf_i, sem_i[1], c1, 1).start()
        dma_in(xt_hbm, buf_t, sem_t[1], c1, 1).start()
        do_chunk(c0, 0)

        @pl.when(p + 1 < _NCH // 2)
        def _():
            dma_in(xi_hbm, buf_i, sem_i[0], c0 + 2, 0).start()
            dma_in(xt_hbm, buf_t, sem_t[0], c0 + 2, 0).start()

        do_chunk(c1, 1)
        return carry

    lax.fori_loop(0, _NCH // 2, pair_body, 0)
    pltpu.sync_copy(out_i, oi_hbm.at[pl.ds(base, _UPT)])
    pltpu.sync_copy(out_t, ot_hbm.at[pl.ds(base, _UPT)])


def _sc_reduce(xi, xt):
    mesh = plsc.VectorSubcoreMesh(core_axis_name="c", subcore_axis_name="s")
    f = functools.partial(
        pl.kernel,
        out_type=(jax.ShapeDtypeStruct((_B, _H), jnp.float32),
                  jax.ShapeDtypeStruct((_B, _H), jnp.float32)),
        mesh=mesh,
        scratch_types=[
            pltpu.VMEM((2, _CU, _N, _H), jnp.float32),
            pltpu.VMEM((2, _CU, _N, _H), jnp.float32),
            pltpu.VMEM((_UPT, _H), jnp.float32),
            pltpu.VMEM((_UPT, _H), jnp.float32),
            pltpu.SemaphoreType.DMA,
            pltpu.SemaphoreType.DMA,
            pltpu.SemaphoreType.DMA,
            pltpu.SemaphoreType.DMA,
        ],
    )(_sc_reduce_body)
    return f(xi, xt)


def _tc_tail_body(feat_ref, si_ref, st_ref, wu_ref, bu_ref, wli_ref, wlt_ref,
                  wri_ref, wrt_ref, bli_ref, blt_ref, out_ref):
    inv_n = jnp.float32(1.0 / _N)
    dn = (((1,), (1,)), ((), ()))
    user = jax.lax.dot_general(feat_ref[...], wu_ref[...], dn,
                               preferred_element_type=jnp.float32)
    user = user + bu_ref[...]
    acc = jax.lax.dot_general(si_ref[...] * inv_n, wli_ref[...], dn,
                              preferred_element_type=jnp.float32)
    acc = acc + jax.lax.dot_general(st_ref[...] * inv_n, wlt_ref[...], dn,
                                    preferred_element_type=jnp.float32)
    wr = wri_ref[...] + wrt_ref[...]
    acc = acc + jax.lax.dot_general(user, wr, dn,
                                    preferred_element_type=jnp.float32)
    acc = acc + bli_ref[...] + blt_ref[...]
    out_ref[...] = jnp.maximum(acc, 0.0)


@jax.jit
def kernel(input_text, input_img, base_text_features, base_img_features,
           W_user, b_user, Wl_img, bl_img, Wr_img, Wl_txt, bl_txt, Wr_txt):
    feat = jnp.concatenate([input_text[:, 0, :], input_img[:, 0, :]], axis=1)
    sum_i, sum_t = _sc_reduce(base_img_features, base_text_features)
    out = pl.pallas_call(
        _tc_tail_body,
        out_shape=jax.ShapeDtypeStruct((_B, _H), jnp.float32),
    )(feat, sum_i, sum_t,
      W_user, b_user.reshape(1, _H), Wl_img, Wl_txt, Wr_img, Wr_txt,
      bl_img.reshape(1, _H), bl_txt.reshape(1, _H))
    return out


# speedup vs baseline: 2.5995x; 1.1450x over previous
import functools

import jax
import jax.numpy as jnp
from jax import lax
from jax.experimental import pallas as pl
from jax.experimental.pallas import tpu as pltpu
from jax.experimental.pallas import tpu_sc as plsc

_B = 2048
_N = 64
_H = 128
_FEAT = 512
_L = 16
_NC = 2
_NS = 16
_NW = _NC * _NS
_HV = _H // _L

_KSC = 768
_KTC = _B - _KSC
_UPT = _KSC // _NW
_CU = 2
_NCH = _UPT // _CU
_RU = 8

_BB = 128
_BB2 = 256


def _sc_reduce_body(xi_hbm, xt_hbm, oi_hbm, ot_hbm, buf_i, buf_t,
                    out_i, out_t, si0, si1, st0, st1):
    wid = lax.axis_index("s") * _NC + lax.axis_index("c")
    base = _KTC + wid * _UPT
    obase = wid * _UPT
    sem_i = (si0, si1)
    sem_t = (st0, st1)

    def dma_in(hbm, buf, sem, c, slot):
        return pltpu.make_async_copy(
            hbm.at[pl.ds(base + c * _CU, _CU)], buf.at[slot], sem)

    def acc_user(buf, slot, u, out_v, row):
        def g_body(g, accs):
            r0 = g * _RU
            for k in range(_RU):
                accs = tuple(accs[h] + buf[slot, u, r0 + k, pl.ds(h * _L, _L)]
                             for h in range(_HV))
            return accs

        accs = lax.fori_loop(
            0, _N // _RU, g_body,
            tuple(jnp.zeros((_L,), jnp.float32) for _ in range(_HV)))
        for h in range(_HV):
            out_v[row, pl.ds(h * _L, _L)] = accs[h]

    def do_chunk(c, slot):
        dma_in(xi_hbm, buf_i, sem_i[slot], c, slot).wait()
        dma_in(xt_hbm, buf_t, sem_t[slot], c, slot).wait()
        for u in range(_CU):
            acc_user(buf_i, slot, u, out_i, c * _CU + u)
            acc_user(buf_t, slot, u, out_t, c * _CU + u)

    dma_in(xi_hbm, buf_i, sem_i[0], 0, 0).start()
    dma_in(xt_hbm, buf_t, sem_t[0], 0, 0).start()

    def pair_body(p, carry):
        c0 = p * 2
        c1 = c0 + 1
        dma_in(xi_hbm, buf_i, sem_i[1], c1, 1).start()
        dma_in(xt_hbm, buf_t, sem_t[1], c1, 1).start()
        do_chunk(c0, 0)

        @pl.when(p + 1 < _NCH // 2)
        def _():
            dma_in(xi_hbm, buf_i, sem_i[0], c0 + 2, 0).start()
            dma_in(xt_hbm, buf_t, sem_t[0], c0 + 2, 0).start()

        do_chunk(c1, 1)
        return carry

    lax.fori_loop(0, _NCH // 2, pair_body, 0)
    pltpu.sync_copy(out_i, oi_hbm.at[pl.ds(obase, _UPT)])
    pltpu.sync_copy(out_t, ot_hbm.at[pl.ds(obase, _UPT)])


def _sc_reduce(xi, xt):
    mesh = plsc.VectorSubcoreMesh(core_axis_name="c", subcore_axis_name="s")
    f = functools.partial(
        pl.kernel,
        out_type=(jax.ShapeDtypeStruct((_KSC, _H), jnp.float32),
                  jax.ShapeDtypeStruct((_KSC, _H), jnp.float32)),
        mesh=mesh,
        scratch_types=[
            pltpu.VMEM((2, _CU, _N, _H), jnp.float32),
            pltpu.VMEM((2, _CU, _N, _H), jnp.float32),
            pltpu.VMEM((_UPT, _H), jnp.float32),
            pltpu.VMEM((_UPT, _H), jnp.float32),
            pltpu.SemaphoreType.DMA,
            pltpu.SemaphoreType.DMA,
            pltpu.SemaphoreType.DMA,
            pltpu.SemaphoreType.DMA,
        ],
    )(_sc_reduce_body)
    return f(xi, xt)


def _tail(user, agg_i, agg_t, wli, wlt, wri, wrt, bli, blt):
    dn = (((1,), (1,)), ((), ()))
    inv_n = jnp.float32(1.0 / _N)
    acc = jax.lax.dot_general(agg_i * inv_n, wli, dn,
                              preferred_element_type=jnp.float32)
    acc = acc + jax.lax.dot_general(agg_t * inv_n, wlt, dn,
                                    preferred_element_type=jnp.float32)
    acc = acc + jax.lax.dot_general(user, wri + wrt, dn,
                                    preferred_element_type=jnp.float32)
    return jnp.maximum(acc + bli + blt, 0.0)


def _user_lin(feat, wu, bu):
    dn = (((1,), (1,)), ((), ()))
    return jax.lax.dot_general(feat, wu, dn,
                               preferred_element_type=jnp.float32) + bu


def _tc_head_body(feat_ref, xi_ref, xt_ref, wu_ref, bu_ref, wli_ref, wlt_ref,
                  wri_ref, wrt_ref, bli_ref, blt_ref, out_ref):
    agg_i = jnp.sum(xi_ref[...], axis=1)
    agg_t = jnp.sum(xt_ref[...], axis=1)
    user = _user_lin(feat_ref[...], wu_ref[...], bu_ref[...])
    out_ref[...] = _tail(user, agg_i, agg_t, wli_ref[...], wlt_ref[...],
                         wri_ref[...], wrt_ref[...], bli_ref[...],
                         blt_ref[...])


def _tc_tail_body(feat_ref, si_ref, st_ref, wu_ref, bu_ref, wli_ref, wlt_ref,
                  wri_ref, wrt_ref, bli_ref, blt_ref, out_ref):
    user = _user_lin(feat_ref[...], wu_ref[...], bu_ref[...])
    out_ref[...] = _tail(user, si_ref[...], st_ref[...], wli_ref[...],
                         wlt_ref[...], wri_ref[...], wrt_ref[...],
                         bli_ref[...], blt_ref[...])


@jax.jit
def kernel(input_text, input_img, base_text_features, base_img_features,
           W_user, b_user, Wl_img, bl_img, Wr_img, Wl_txt, bl_txt, Wr_txt):
    feat = jnp.concatenate([input_text[:, 0, :], input_img[:, 0, :]], axis=1)
    weights = (W_user, b_user.reshape(1, _H), Wl_img, Wl_txt, Wr_img, Wr_txt,
               bl_img.reshape(1, _H), bl_txt.reshape(1, _H))
    full = lambda shape: pl.BlockSpec(shape, lambda i: (0,) * len(shape))
    wspecs = [full((_H, _FEAT)), full((1, _H)), full((_H, _H)),
              full((_H, _H)), full((_H, _H)), full((_H, _H)),
              full((1, _H)), full((1, _H))]

    sum_i, sum_t = _sc_reduce(base_img_features, base_text_features)

    out_head = pl.pallas_call(
        _tc_head_body,
        grid=(_KTC // _BB,),
        in_specs=[
            pl.BlockSpec((_BB, _FEAT), lambda i: (i, 0)),
            pl.BlockSpec((_BB, _N, _H), lambda i: (i, 0, 0)),
            pl.BlockSpec((_BB, _N, _H), lambda i: (i, 0, 0)),
            *wspecs,
        ],
        out_specs=pl.BlockSpec((_BB, _H), lambda i: (i, 0)),
        out_shape=jax.ShapeDtypeStruct((_KTC, _H), jnp.float32),
    )(feat, base_img_features, base_text_features, *weights)

    off = _KTC // _BB2
    out_tail = pl.pallas_call(
        _tc_tail_body,
        grid=(_KSC // _BB2,),
        in_specs=[
            pl.BlockSpec((_BB2, _FEAT), lambda i: (i + off, 0)),
            pl.BlockSpec((_BB2, _H), lambda i: (i, 0)),
            pl.BlockSpec((_BB2, _H), lambda i: (i, 0)),
            *wspecs,
        ],
        out_specs=pl.BlockSpec((_BB2, _H), lambda i: (i, 0)),
        out_shape=jax.ShapeDtypeStruct((_KSC, _H), jnp.float32),
    )(feat, sum_i, sum_t, *weights)

    return jnp.concatenate([out_head, out_tail], axis=0)


# speedup vs baseline: 2.6026x; 1.0012x over previous
import functools

import jax
import jax.numpy as jnp
from jax import lax
from jax.experimental import pallas as pl
from jax.experimental.pallas import tpu as pltpu
from jax.experimental.pallas import tpu_sc as plsc

_B = 2048
_N = 64
_H = 128
_FEAT = 512
_L = 16
_NC = 2
_NS = 16
_NW = _NC * _NS
_HV = _H // _L

_KSC = 768
_KTC = _B - _KSC
_UPT = _KSC // _NW
_CU = 2
_NCH = _UPT // _CU
_RU = 8

_BB = 128
_BB2 = 256


def _sc_reduce_body(xi_hbm, xt_hbm, oi_hbm, ot_hbm, buf_i, buf_t,
                    out_i, out_t, si0, si1, st0, st1):
    wid = lax.axis_index("s") * _NC + lax.axis_index("c")
    base = _KTC + wid * _UPT
    obase = wid * _UPT
    sem_i = (si0, si1)
    sem_t = (st0, st1)

    def dma_in(hbm, buf, sem, c, slot):
        return pltpu.make_async_copy(
            hbm.at[pl.ds(base + c * _CU, _CU)], buf.at[slot], sem)

    def acc_user(buf, slot, u, out_v, row):
        def g_body(g, accs):
            r0 = g * _RU
            for k in range(_RU):
                accs = tuple(accs[h] + buf[slot, u, r0 + k, pl.ds(h * _L, _L)]
                             for h in range(_HV))
            return accs

        accs = lax.fori_loop(
            0, _N // _RU, g_body,
            tuple(jnp.zeros((_L,), jnp.float32) for _ in range(_HV)))
        for h in range(_HV):
            out_v[row, pl.ds(h * _L, _L)] = accs[h]

    def do_chunk(c, slot):
        dma_in(xi_hbm, buf_i, sem_i[slot], c, slot).wait()
        dma_in(xt_hbm, buf_t, sem_t[slot], c, slot).wait()
        for u in range(_CU):
            acc_user(buf_i, slot, u, out_i, c * _CU + u)
            acc_user(buf_t, slot, u, out_t, c * _CU + u)

    dma_in(xi_hbm, buf_i, sem_i[0], 0, 0).start()
    dma_in(xt_hbm, buf_t, sem_t[0], 0, 0).start()

    def pair_body(p, carry):
        c0 = p * 2
        c1 = c0 + 1
        dma_in(xi_hbm, buf_i, sem_i[1], c1, 1).start()
        dma_in(xt_hbm, buf_t, sem_t[1], c1, 1).start()
        do_chunk(c0, 0)

        @pl.when(p + 1 < _NCH // 2)
        def _():
            dma_in(xi_hbm, buf_i, sem_i[0], c0 + 2, 0).start()
            dma_in(xt_hbm, buf_t, sem_t[0], c0 + 2, 0).start()

        do_chunk(c1, 1)
        return carry

    lax.fori_loop(0, _NCH // 2, pair_body, 0)
    pltpu.sync_copy(out_i, oi_hbm.at[pl.ds(obase, _UPT)])
    pltpu.sync_copy(out_t, ot_hbm.at[pl.ds(obase, _UPT)])


def _sc_reduce(xi, xt):
    mesh = plsc.VectorSubcoreMesh(core_axis_name="c", subcore_axis_name="s")
    f = functools.partial(
        pl.kernel,
        out_type=(jax.ShapeDtypeStruct((_KSC, _H), jnp.float32),
                  jax.ShapeDtypeStruct((_KSC, _H), jnp.float32)),
        mesh=mesh,
        scratch_types=[
            pltpu.VMEM((2, _CU, _N, _H), jnp.float32),
            pltpu.VMEM((2, _CU, _N, _H), jnp.float32),
            pltpu.VMEM((_UPT, _H), jnp.float32),
            pltpu.VMEM((_UPT, _H), jnp.float32),
            pltpu.SemaphoreType.DMA,
            pltpu.SemaphoreType.DMA,
            pltpu.SemaphoreType.DMA,
            pltpu.SemaphoreType.DMA,
        ],
    )(_sc_reduce_body)
    return f(xi, xt)


def _tail(user, agg_i, agg_t, wli, wlt, wri, wrt, bli, blt):
    dn = (((1,), (1,)), ((), ()))
    inv_n = jnp.float32(1.0 / _N)
    acc = jax.lax.dot_general(agg_i * inv_n, wli, dn,
                              preferred_element_type=jnp.float32)
    acc = acc + jax.lax.dot_general(agg_t * inv_n, wlt, dn,
                                    preferred_element_type=jnp.float32)
    acc = acc + jax.lax.dot_general(user, wri + wrt, dn,
                                    preferred_element_type=jnp.float32)
    return jnp.maximum(acc + bli + blt, 0.0)


def _user_lin(feat, wu, bu):
    dn = (((1,), (1,)), ((), ()))
    return jax.lax.dot_general(feat, wu, dn,
                               preferred_element_type=jnp.float32) + bu


def _tc_head_body(feat_ref, xi_ref, xt_ref, wu_ref, bu_ref, wli_ref, wlt_ref,
                  wri_ref, wrt_ref, bli_ref, blt_ref, out_ref):
    agg_i = jnp.sum(xi_ref[...], axis=1)
    agg_t = jnp.sum(xt_ref[...], axis=1)
    user = _user_lin(feat_ref[...], wu_ref[...], bu_ref[...])
    out_ref[...] = _tail(user, agg_i, agg_t, wli_ref[...], wlt_ref[...],
                         wri_ref[...], wrt_ref[...], bli_ref[...],
                         blt_ref[...])


def _tc_tail_body(feat_ref, si_ref, st_ref, wu_ref, bu_ref, wli_ref, wlt_ref,
                  wri_ref, wrt_ref, bli_ref, blt_ref, out_ref):
    user = _user_lin(feat_ref[...], wu_ref[...], bu_ref[...])
    out_ref[...] = _tail(user, si_ref[...], st_ref[...], wli_ref[...],
                         wlt_ref[...], wri_ref[...], wrt_ref[...],
                         bli_ref[...], blt_ref[...])


@jax.jit
def kernel(input_text, input_img, base_text_features, base_img_features,
           W_user, b_user, Wl_img, bl_img, Wr_img, Wl_txt, bl_txt, Wr_txt):
    feat = jnp.concatenate([input_text[:, 0, :], input_img[:, 0, :]], axis=1)
    weights = (W_user, b_user.reshape(1, _H), Wl_img, Wl_txt, Wr_img, Wr_txt,
               bl_img.reshape(1, _H), bl_txt.reshape(1, _H))
    full = lambda shape: pl.BlockSpec(shape, lambda i: (0,) * len(shape))
    wspecs = [full((_H, _FEAT)), full((1, _H)), full((_H, _H)),
              full((_H, _H)), full((_H, _H)), full((_H, _H)),
              full((1, _H)), full((1, _H))]

    out_head = pl.pallas_call(
        _tc_head_body,
        grid=(_KTC // _BB,),
        in_specs=[
            pl.BlockSpec((_BB, _FEAT), lambda i: (i, 0)),
            pl.BlockSpec((_BB, _N, _H), lambda i: (i, 0, 0)),
            pl.BlockSpec((_BB, _N, _H), lambda i: (i, 0, 0)),
            *wspecs,
        ],
        out_specs=pl.BlockSpec((_BB, _H), lambda i: (i, 0)),
        out_shape=jax.ShapeDtypeStruct((_KTC, _H), jnp.float32),
    )(feat, base_img_features, base_text_features, *weights)

    sum_i, sum_t = _sc_reduce(base_img_features, base_text_features)

    off = _KTC // _BB2
    out_tail = pl.pallas_call(
        _tc_tail_body,
        grid=(_KSC // _BB2,),
        in_specs=[
            pl.BlockSpec((_BB2, _FEAT), lambda i: (i + off, 0)),
            pl.BlockSpec((_BB2, _H), lambda i: (i, 0)),
            pl.BlockSpec((_BB2, _H), lambda i: (i, 0)),
            *wspecs,
        ],
        out_specs=pl.BlockSpec((_BB2, _H), lambda i: (i, 0)),
        out_shape=jax.ShapeDtypeStruct((_KSC, _H), jnp.float32),
    )(feat, sum_i, sum_t, *weights)

    return jnp.concatenate([out_head, out_tail], axis=0)


# speedup vs baseline: 2.7020x; 1.0382x over previous
import functools

import jax
import jax.numpy as jnp
from jax import lax
from jax.experimental import pallas as pl
from jax.experimental.pallas import tpu as pltpu
from jax.experimental.pallas import tpu_sc as plsc

_B = 2048
_N = 64
_H = 128
_FEAT = 512
_L = 16
_NC = 2
_NS = 16
_NW = _NC * _NS
_HV = _H // _L

_KSC = 512
_KTC = _B - _KSC
_UPT = _KSC // _NW
_CU = 2
_NCH = _UPT // _CU
_RU = 8

_BB = 128
_BB2 = 512


def _sc_reduce_body(xi_hbm, xt_hbm, oi_hbm, ot_hbm, buf_i, buf_t,
                    out_i, out_t, si0, si1, st0, st1):
    wid = lax.axis_index("s") * _NC + lax.axis_index("c")
    base = _KTC + wid * _UPT
    obase = wid * _UPT
    sem_i = (si0, si1)
    sem_t = (st0, st1)

    def dma_in(hbm, buf, sem, c, slot):
        return pltpu.make_async_copy(
            hbm.at[pl.ds(base + c * _CU, _CU)], buf.at[slot], sem)

    def acc_user(buf, slot, u, out_v, row):
        def g_body(g, accs):
            r0 = g * _RU
            for k in range(_RU):
                accs = tuple(accs[h] + buf[slot, u, r0 + k, pl.ds(h * _L, _L)]
                             for h in range(_HV))
            return accs

        accs = lax.fori_loop(
            0, _N // _RU, g_body,
            tuple(jnp.zeros((_L,), jnp.float32) for _ in range(_HV)))
        for h in range(_HV):
            out_v[row, pl.ds(h * _L, _L)] = accs[h]

    def do_chunk(c, slot):
        dma_in(xi_hbm, buf_i, sem_i[slot], c, slot).wait()
        dma_in(xt_hbm, buf_t, sem_t[slot], c, slot).wait()
        for u in range(_CU):
            acc_user(buf_i, slot, u, out_i, c * _CU + u)
            acc_user(buf_t, slot, u, out_t, c * _CU + u)

    dma_in(xi_hbm, buf_i, sem_i[0], 0, 0).start()
    dma_in(xt_hbm, buf_t, sem_t[0], 0, 0).start()

    def pair_body(p, carry):
        c0 = p * 2
        c1 = c0 + 1
        dma_in(xi_hbm, buf_i, sem_i[1], c1, 1).start()
        dma_in(xt_hbm, buf_t, sem_t[1], c1, 1).start()
        do_chunk(c0, 0)

        @pl.when(p + 1 < _NCH // 2)
        def _():
            dma_in(xi_hbm, buf_i, sem_i[0], c0 + 2, 0).start()
            dma_in(xt_hbm, buf_t, sem_t[0], c0 + 2, 0).start()

        do_chunk(c1, 1)
        return carry

    lax.fori_loop(0, _NCH // 2, pair_body, 0)
    pltpu.sync_copy(out_i, oi_hbm.at[pl.ds(obase, _UPT)])
    pltpu.sync_copy(out_t, ot_hbm.at[pl.ds(obase, _UPT)])


def _sc_reduce(xi, xt):
    mesh = plsc.VectorSubcoreMesh(core_axis_name="c", subcore_axis_name="s")
    f = functools.partial(
        pl.kernel,
        out_type=(jax.ShapeDtypeStruct((_KSC, _H), jnp.float32),
                  jax.ShapeDtypeStruct((_KSC, _H), jnp.float32)),
        mesh=mesh,
        scratch_types=[
            pltpu.VMEM((2, _CU, _N, _H), jnp.float32),
            pltpu.VMEM((2, _CU, _N, _H), jnp.float32),
            pltpu.VMEM((_UPT, _H), jnp.float32),
            pltpu.VMEM((_UPT, _H), jnp.float32),
            pltpu.SemaphoreType.DMA,
            pltpu.SemaphoreType.DMA,
            pltpu.SemaphoreType.DMA,
            pltpu.SemaphoreType.DMA,
        ],
    )(_sc_reduce_body)
    return f(xi, xt)


def _tail(user, agg_i, agg_t, wli, wlt, wri, wrt, bli, blt):
    dn = (((1,), (1,)), ((), ()))
    inv_n = jnp.float32(1.0 / _N)
    acc = jax.lax.dot_general(agg_i * inv_n, wli, dn,
                              preferred_element_type=jnp.float32)
    acc = acc + jax.lax.dot_general(agg_t * inv_n, wlt, dn,
                                    preferred_element_type=jnp.float32)
    acc = acc + jax.lax.dot_general(user, wri + wrt, dn,
                                    preferred_element_type=jnp.float32)
    return jnp.maximum(acc + bli + blt, 0.0)


def _user_lin(feat, wu, bu):
    dn = (((1,), (1,)), ((), ()))
    return jax.lax.dot_general(feat, wu, dn,
                               preferred_element_type=jnp.float32) + bu


def _tc_head_body(feat_ref, xi_ref, xt_ref, wu_ref, bu_ref, wli_ref, wlt_ref,
                  wri_ref, wrt_ref, bli_ref, blt_ref, out_ref):
    agg_i = jnp.sum(xi_ref[...], axis=1)
    agg_t = jnp.sum(xt_ref[...], axis=1)
    user = _user_lin(feat_ref[...], wu_ref[...], bu_ref[...])
    out_ref[...] = _tail(user, agg_i, agg_t, wli_ref[...], wlt_ref[...],
                         wri_ref[...], wrt_ref[...], bli_ref[...],
                         blt_ref[...])


def _tc_tail_body(feat_ref, si_ref, st_ref, wu_ref, bu_ref, wli_ref, wlt_ref,
                  wri_ref, wrt_ref, bli_ref, blt_ref, out_ref):
    user = _user_lin(feat_ref[...], wu_ref[...], bu_ref[...])
    out_ref[...] = _tail(user, si_ref[...], st_ref[...], wli_ref[...],
                         wlt_ref[...], wri_ref[...], wrt_ref[...],
                         bli_ref[...], blt_ref[...])


@jax.jit
def kernel(input_text, input_img, base_text_features, base_img_features,
           W_user, b_user, Wl_img, bl_img, Wr_img, Wl_txt, bl_txt, Wr_txt):
    feat = jnp.concatenate([input_text[:, 0, :], input_img[:, 0, :]], axis=1)
    weights = (W_user, b_user.reshape(1, _H), Wl_img, Wl_txt, Wr_img, Wr_txt,
               bl_img.reshape(1, _H), bl_txt.reshape(1, _H))
    full = lambda shape: pl.BlockSpec(shape, lambda i: (0,) * len(shape))
    wspecs = [full((_H, _FEAT)), full((1, _H)), full((_H, _H)),
              full((_H, _H)), full((_H, _H)), full((_H, _H)),
              full((1, _H)), full((1, _H))]

    out_head = pl.pallas_call(
        _tc_head_body,
        grid=(_KTC // _BB,),
        in_specs=[
            pl.BlockSpec((_BB, _FEAT), lambda i: (i, 0)),
            pl.BlockSpec((_BB, _N, _H), lambda i: (i, 0, 0)),
            pl.BlockSpec((_BB, _N, _H), lambda i: (i, 0, 0)),
            *wspecs,
        ],
        out_specs=pl.BlockSpec((_BB, _H), lambda i: (i, 0)),
        out_shape=jax.ShapeDtypeStruct((_KTC, _H), jnp.float32),
    )(feat, base_img_features, base_text_features, *weights)

    sum_i, sum_t = _sc_reduce(base_img_features, base_text_features)

    off = _KTC // _BB2
    out_tail = pl.pallas_call(
        _tc_tail_body,
        grid=(_KSC // _BB2,),
        in_specs=[
            pl.BlockSpec((_BB2, _FEAT), lambda i: (i + off, 0)),
            pl.BlockSpec((_BB2, _H), lambda i: (i, 0)),
            pl.BlockSpec((_BB2, _H), lambda i: (i, 0)),
            *wspecs,
        ],
        out_specs=pl.BlockSpec((_BB2, _H), lambda i: (i, 0)),
        out_shape=jax.ShapeDtypeStruct((_KSC, _H), jnp.float32),
    )(feat, sum_i, sum_t, *weights)

    return jnp.concatenate([out_head, out_tail], axis=0)


# speedup vs baseline: 2.7438x; 1.0155x over previous
import functools

import jax
import jax.numpy as jnp
from jax import lax
from jax.experimental import pallas as pl
from jax.experimental.pallas import tpu as pltpu
from jax.experimental.pallas import tpu_sc as plsc

_B = 2048
_N = 64
_H = 128
_FEAT = 512
_L = 16
_NC = 2
_NS = 16
_NW = _NC * _NS
_HV = _H // _L

_KSC = 256
_KTC = _B - _KSC
_UPT = _KSC // _NW
_CU = 2
_NCH = _UPT // _CU
_RU = 8

_BB = 128
_BB2 = 256


def _sc_reduce_body(xi_hbm, xt_hbm, oi_hbm, ot_hbm, buf_i, buf_t,
                    out_i, out_t, si0, si1, st0, st1):
    wid = lax.axis_index("s") * _NC + lax.axis_index("c")
    base = _KTC + wid * _UPT
    obase = wid * _UPT
    sem_i = (si0, si1)
    sem_t = (st0, st1)

    def dma_in(hbm, buf, sem, c, slot):
        return pltpu.make_async_copy(
            hbm.at[pl.ds(base + c * _CU, _CU)], buf.at[slot], sem)

    def acc_user(buf, slot, u, out_v, row):
        def g_body(g, accs):
            r0 = g * _RU
            for k in range(_RU):
                accs = tuple(accs[h] + buf[slot, u, r0 + k, pl.ds(h * _L, _L)]
                             for h in range(_HV))
            return accs

        accs = lax.fori_loop(
            0, _N // _RU, g_body,
            tuple(jnp.zeros((_L,), jnp.float32) for _ in range(_HV)))
        for h in range(_HV):
            out_v[row, pl.ds(h * _L, _L)] = accs[h]

    def do_chunk(c, slot):
        dma_in(xi_hbm, buf_i, sem_i[slot], c, slot).wait()
        dma_in(xt_hbm, buf_t, sem_t[slot], c, slot).wait()
        for u in range(_CU):
            acc_user(buf_i, slot, u, out_i, c * _CU + u)
            acc_user(buf_t, slot, u, out_t, c * _CU + u)

    dma_in(xi_hbm, buf_i, sem_i[0], 0, 0).start()
    dma_in(xt_hbm, buf_t, sem_t[0], 0, 0).start()

    def pair_body(p, carry):
        c0 = p * 2
        c1 = c0 + 1
        dma_in(xi_hbm, buf_i, sem_i[1], c1, 1).start()
        dma_in(xt_hbm, buf_t, sem_t[1], c1, 1).start()
        do_chunk(c0, 0)

        @pl.when(p + 1 < _NCH // 2)
        def _():
            dma_in(xi_hbm, buf_i, sem_i[0], c0 + 2, 0).start()
            dma_in(xt_hbm, buf_t, sem_t[0], c0 + 2, 0).start()

        do_chunk(c1, 1)
        return carry

    lax.fori_loop(0, _NCH // 2, pair_body, 0)
    pltpu.sync_copy(out_i, oi_hbm.at[pl.ds(obase, _UPT)])
    pltpu.sync_copy(out_t, ot_hbm.at[pl.ds(obase, _UPT)])


def _sc_reduce(xi, xt):
    mesh = plsc.VectorSubcoreMesh(core_axis_name="c", subcore_axis_name="s")
    f = functools.partial(
        pl.kernel,
        out_type=(jax.ShapeDtypeStruct((_KSC, _H), jnp.float32),
                  jax.ShapeDtypeStruct((_KSC, _H), jnp.float32)),
        mesh=mesh,
        scratch_types=[
            pltpu.VMEM((2, _CU, _N, _H), jnp.float32),
            pltpu.VMEM((2, _CU, _N, _H), jnp.float32),
            pltpu.VMEM((_UPT, _H), jnp.float32),
            pltpu.VMEM((_UPT, _H), jnp.float32),
            pltpu.SemaphoreType.DMA,
            pltpu.SemaphoreType.DMA,
            pltpu.SemaphoreType.DMA,
            pltpu.SemaphoreType.DMA,
        ],
    )(_sc_reduce_body)
    return f(xi, xt)


def _tail(user, agg_i, agg_t, wli, wlt, wri, wrt, bli, blt):
    dn = (((1,), (1,)), ((), ()))
    inv_n = jnp.float32(1.0 / _N)
    acc = jax.lax.dot_general(agg_i * inv_n, wli, dn,
                              preferred_element_type=jnp.float32)
    acc = acc + jax.lax.dot_general(agg_t * inv_n, wlt, dn,
                                    preferred_element_type=jnp.float32)
    acc = acc + jax.lax.dot_general(user, wri + wrt, dn,
                                    preferred_element_type=jnp.float32)
    return jnp.maximum(acc + bli + blt, 0.0)


def _user_lin(feat, wu, bu):
    dn = (((1,), (1,)), ((), ()))
    return jax.lax.dot_general(feat, wu, dn,
                               preferred_element_type=jnp.float32) + bu


def _tc_head_body(feat_ref, xi_ref, xt_ref, wu_ref, bu_ref, wli_ref, wlt_ref,
                  wri_ref, wrt_ref, bli_ref, blt_ref, out_ref):
    agg_i = jnp.sum(xi_ref[...], axis=1)
    agg_t = jnp.sum(xt_ref[...], axis=1)
    user = _user_lin(feat_ref[...], wu_ref[...], bu_ref[...])
    out_ref[...] = _tail(user, agg_i, agg_t, wli_ref[...], wlt_ref[...],
                         wri_ref[...], wrt_ref[...], bli_ref[...],
                         blt_ref[...])


def _tc_tail_body(feat_ref, si_ref, st_ref, wu_ref, bu_ref, wli_ref, wlt_ref,
                  wri_ref, wrt_ref, bli_ref, blt_ref, out_ref):
    user = _user_lin(feat_ref[...], wu_ref[...], bu_ref[...])
    out_ref[...] = _tail(user, si_ref[...], st_ref[...], wli_ref[...],
                         wlt_ref[...], wri_ref[...], wrt_ref[...],
                         bli_ref[...], blt_ref[...])


@jax.jit
def kernel(input_text, input_img, base_text_features, base_img_features,
           W_user, b_user, Wl_img, bl_img, Wr_img, Wl_txt, bl_txt, Wr_txt):
    feat = jnp.concatenate([input_text[:, 0, :], input_img[:, 0, :]], axis=1)
    weights = (W_user, b_user.reshape(1, _H), Wl_img, Wl_txt, Wr_img, Wr_txt,
               bl_img.reshape(1, _H), bl_txt.reshape(1, _H))
    full = lambda shape: pl.BlockSpec(shape, lambda i: (0,) * len(shape))
    wspecs = [full((_H, _FEAT)), full((1, _H)), full((_H, _H)),
              full((_H, _H)), full((_H, _H)), full((_H, _H)),
              full((1, _H)), full((1, _H))]

    out_head = pl.pallas_call(
        _tc_head_body,
        grid=(_KTC // _BB,),
        in_specs=[
            pl.BlockSpec((_BB, _FEAT), lambda i: (i, 0)),
            pl.BlockSpec((_BB, _N, _H), lambda i: (i, 0, 0)),
            pl.BlockSpec((_BB, _N, _H), lambda i: (i, 0, 0)),
            *wspecs,
        ],
        out_specs=pl.BlockSpec((_BB, _H), lambda i: (i, 0)),
        out_shape=jax.ShapeDtypeStruct((_KTC, _H), jnp.float32),
    )(feat, base_img_features, base_text_features, *weights)

    sum_i, sum_t = _sc_reduce(base_img_features, base_text_features)

    off = _KTC // _BB2
    out_tail = pl.pallas_call(
        _tc_tail_body,
        grid=(_KSC // _BB2,),
        in_specs=[
            pl.BlockSpec((_BB2, _FEAT), lambda i: (i + off, 0)),
            pl.BlockSpec((_BB2, _H), lambda i: (i, 0)),
            pl.BlockSpec((_BB2, _H), lambda i: (i, 0)),
            *wspecs,
        ],
        out_specs=pl.BlockSpec((_BB2, _H), lambda i: (i, 0)),
        out_shape=jax.ShapeDtypeStruct((_KSC, _H), jnp.float32),
    )(feat, sum_i, sum_t, *weights)

    return jnp.concatenate([out_head, out_tail], axis=0)


# speedup vs baseline: 3.6035x; 1.3133x over previous
import jax
import jax.numpy as jnp
from jax.experimental import pallas as pl

_B = 2048
_N = 64
_H = 128
_FEAT = 512
_FH = _FEAT // 2
_BB = 128


def _tc_body(it_ref, ii_ref, xi_ref, xt_ref, wut_ref, wui_ref, bu_ref,
             wli_ref, wlt_ref, wr_ref, bsum_ref, out_ref):
    inv_n = jnp.float32(1.0 / _N)
    agg_i = jnp.sum(xi_ref[...], axis=1) * inv_n
    agg_t = jnp.sum(xt_ref[...], axis=1) * inv_n
    dn = (((1,), (1,)), ((), ()))
    user = jax.lax.dot_general(it_ref[...], wut_ref[...], dn,
                               preferred_element_type=jnp.float32)
    user = user + jax.lax.dot_general(ii_ref[...], wui_ref[...], dn,
                                      preferred_element_type=jnp.float32)
    user = user + bu_ref[...]
    acc = jax.lax.dot_general(agg_i, wli_ref[...], dn,
                              preferred_element_type=jnp.float32)
    acc = acc + jax.lax.dot_general(agg_t, wlt_ref[...], dn,
                                    preferred_element_type=jnp.float32)
    acc = acc + jax.lax.dot_general(user, wr_ref[...], dn,
                                    preferred_element_type=jnp.float32)
    out_ref[...] = jnp.maximum(acc + bsum_ref[...], 0.0)


@jax.jit
def kernel(input_text, input_img, base_text_features, base_img_features,
           W_user, b_user, Wl_img, bl_img, Wr_img, Wl_txt, bl_txt, Wr_txt):
    it = input_text.reshape(_B, _FH)
    ii = input_img.reshape(_B, _FH)
    wut = W_user[:, :_FH]
    wui = W_user[:, _FH:]
    wr = Wr_img + Wr_txt
    bsum = (bl_img + bl_txt).reshape(1, _H)
    grid = (_B // _BB,)
    full = lambda shape: pl.BlockSpec(shape, lambda i: (0,) * len(shape))
    out = pl.pallas_call(
        _tc_body,
        grid=grid,
        in_specs=[
            pl.BlockSpec((_BB, _FH), lambda i: (i, 0)),
            pl.BlockSpec((_BB, _FH), lambda i: (i, 0)),
            pl.BlockSpec((_BB, _N, _H), lambda i: (i, 0, 0)),
            pl.BlockSpec((_BB, _N, _H), lambda i: (i, 0, 0)),
            full((_H, _FH)),
            full((_H, _FH)),
            full((1, _H)),
            full((_H, _H)),
            full((_H, _H)),
            full((_H, _H)),
            full((1, _H)),
        ],
        out_specs=pl.BlockSpec((_BB, _H), lambda i: (i, 0)),
        out_shape=jax.ShapeDtypeStruct((_B, _H), jnp.float32),
    )(it, ii, base_img_features, base_text_features,
      wut, wui, b_user.reshape(1, _H), Wl_img, Wl_txt, wr, bsum)
    return out


# speedup vs baseline: 4.1913x; 1.1631x over previous
import jax
import jax.numpy as jnp
from jax.experimental import pallas as pl

_B = 2048
_N = 64
_H = 128
_FEAT = 512
_FH = _FEAT // 2
_BB = 128


def _tc_body(it_ref, ii_ref, xi_ref, xt_ref, wut_ref, wui_ref, bu_ref,
             wli_ref, wlt_ref, wri_ref, wrt_ref, bli_ref, blt_ref, out_ref):
    inv_n = jnp.float32(1.0 / _N)
    agg_i = jnp.sum(xi_ref[...], axis=1) * inv_n
    agg_t = jnp.sum(xt_ref[...], axis=1) * inv_n
    dn = (((1,), (1,)), ((), ()))
    it = it_ref[...].reshape(_BB, _FH)
    ii = ii_ref[...].reshape(_BB, _FH)
    user = jax.lax.dot_general(it, wut_ref[...], dn,
                               preferred_element_type=jnp.float32)
    user = user + jax.lax.dot_general(ii, wui_ref[...], dn,
                                      preferred_element_type=jnp.float32)
    user = user + bu_ref[...]
    acc = jax.lax.dot_general(agg_i, wli_ref[...], dn,
                              preferred_element_type=jnp.float32)
    acc = acc + jax.lax.dot_general(agg_t, wlt_ref[...], dn,
                                    preferred_element_type=jnp.float32)
    wr = wri_ref[...] + wrt_ref[...]
    acc = acc + jax.lax.dot_general(user, wr, dn,
                                    preferred_element_type=jnp.float32)
    out_ref[...] = jnp.maximum(acc + bli_ref[...] + blt_ref[...], 0.0)


@jax.jit
def kernel(input_text, input_img, base_text_features, base_img_features,
           W_user, b_user, Wl_img, bl_img, Wr_img, Wl_txt, bl_txt, Wr_txt):
    wut = W_user[:, :_FH]
    wui = W_user[:, _FH:]
    grid = (_B // _BB,)
    full = lambda shape: pl.BlockSpec(shape, lambda i: (0,) * len(shape))
    out = pl.pallas_call(
        _tc_body,
        grid=grid,
        in_specs=[
            pl.BlockSpec((_BB, 1, _FH), lambda i: (i, 0, 0)),
            pl.BlockSpec((_BB, 1, _FH), lambda i: (i, 0, 0)),
            pl.BlockSpec((_BB, _N, _H), lambda i: (i, 0, 0)),
            pl.BlockSpec((_BB, _N, _H), lambda i: (i, 0, 0)),
            full((_H, _FH)),
            full((_H, _FH)),
            full((1, _H)),
            full((_H, _H)),
            full((_H, _H)),
            full((_H, _H)),
            full((_H, _H)),
            full((1, _H)),
            full((1, _H)),
        ],
        out_specs=pl.BlockSpec((_BB, _H), lambda i: (i, 0)),
        out_shape=jax.ShapeDtypeStruct((_B, _H), jnp.float32),
    )(input_text, input_img, base_img_features, base_text_features,
      wut, wui, b_user.reshape(1, _H), Wl_img, Wl_txt, Wr_img, Wr_txt,
      bl_img.reshape(1, _H), bl_txt.reshape(1, _H))
    return out


# speedup vs baseline: 4.3688x; 1.0423x over previous
import jax
import jax.numpy as jnp
from jax.experimental import pallas as pl

_B = 2048
_N = 64
_H = 128
_FEAT = 512
_FH = _FEAT // 2
_BB = 128


def _tc_body(it_ref, ii_ref, xi_ref, xt_ref, wu_ref, bu_ref,
             wli_ref, wlt_ref, wri_ref, wrt_ref, bli_ref, blt_ref, out_ref):
    inv_n = jnp.float32(1.0 / _N)
    agg_i = jnp.sum(xi_ref[...], axis=1) * inv_n
    agg_t = jnp.sum(xt_ref[...], axis=1) * inv_n
    dn = (((1,), (1,)), ((), ()))
    it = it_ref[...].reshape(_BB, _FH)
    ii = ii_ref[...].reshape(_BB, _FH)
    wu = wu_ref[...]
    user = jax.lax.dot_general(it, wu[:, :_FH], dn,
                               preferred_element_type=jnp.float32)
    user = user + jax.lax.dot_general(ii, wu[:, _FH:], dn,
                                      preferred_element_type=jnp.float32)
    user = user + bu_ref[...]
    acc = jax.lax.dot_general(agg_i, wli_ref[...], dn,
                              preferred_element_type=jnp.float32)
    acc = acc + jax.lax.dot_general(agg_t, wlt_ref[...], dn,
                                    preferred_element_type=jnp.float32)
    wr = wri_ref[...] + wrt_ref[...]
    acc = acc + jax.lax.dot_general(user, wr, dn,
                                    preferred_element_type=jnp.float32)
    out_ref[...] = jnp.maximum(acc + bli_ref[...] + blt_ref[...], 0.0)


@jax.jit
def kernel(input_text, input_img, base_text_features, base_img_features,
           W_user, b_user, Wl_img, bl_img, Wr_img, Wl_txt, bl_txt, Wr_txt):
    grid = (_B // _BB,)
    full = lambda shape: pl.BlockSpec(shape, lambda i: (0,) * len(shape))
    out = pl.pallas_call(
        _tc_body,
        grid=grid,
        in_specs=[
            pl.BlockSpec((_BB, 1, _FH), lambda i: (i, 0, 0)),
            pl.BlockSpec((_BB, 1, _FH), lambda i: (i, 0, 0)),
            pl.BlockSpec((_BB, _N, _H), lambda i: (i, 0, 0)),
            pl.BlockSpec((_BB, _N, _H), lambda i: (i, 0, 0)),
            full((_H, _FEAT)),
            full((1, _H)),
            full((_H, _H)),
            full((_H, _H)),
            full((_H, _H)),
            full((_H, _H)),
            full((1, _H)),
            full((1, _H)),
        ],
        out_specs=pl.BlockSpec((_BB, _H), lambda i: (i, 0)),
        out_shape=jax.ShapeDtypeStruct((_B, _H), jnp.float32),
    )(input_text, input_img, base_img_features, base_text_features,
      W_user, b_user.reshape(1, _H), Wl_img, Wl_txt, Wr_img, Wr_txt,
      bl_img.reshape(1, _H), bl_txt.reshape(1, _H))
    return out
